# baseline (speedup 1.0000x reference)
import jax
import jax.numpy as jnp
import numpy as np
from jax import lax
from jax.experimental import pallas as pl
from jax.experimental.pallas import tpu as pltpu
from jax.experimental.pallas import tpu_sc as plsc

NC = 2
NS = 16
NW = NC * NS
CH = 128


GRP = 8


NBUF = 2


def _zero_rows(buf, n_sub, d):
  @pl.loop(0, n_sub * 16)
  def _row(i):
    for k in range(d // 16):
      buf[i, pl.ds(k * 16, 16)] = jnp.zeros((16,), jnp.float32)


def _zero_share(buf, acc, sid, rpt, d):
  nfull, tail = divmod(rpt, CH)
  for t in range(nfull):
    pltpu.sync_copy(buf, acc.at[pl.ds(sid * rpt + t * CH, CH)])
  if tail:
    pltpu.sync_copy(buf.at[pl.ds(0, tail)],
                    acc.at[pl.ds(sid * rpt + nfull * CH, tail)])


def _make_edge_pass(n_nodes, n_pad, d, c0, c1):
  rpt = n_pad // NS
  mesh = plsc.VectorSubcoreMesh(core_axis_name="c", subcore_axis_name="s")

  out_type = jax.ShapeDtypeStruct((NC * n_pad, d), jnp.float32)

  scratch = [
      pltpu.VMEM((2, GRP, CH), jnp.int32),
      pltpu.VMEM((2, GRP, CH), jnp.int32),
      [pltpu.VMEM((CH, d), jnp.float32) for _ in range(NBUF)],
      pltpu.VMEM_SHARED((n_pad, d), jnp.float32),
      [pltpu.SemaphoreType.DMA for _ in range(NBUF)],
      [pltpu.SemaphoreType.DMA for _ in range(NBUF)],
  ]

  def body(h_hbm, src_hbm, dst_hbm, part_hbm, sidx, didx, rows, acc, gsem,
           ssem):
    cid = lax.axis_index("c")
    sid = lax.axis_index("s")
    wid = sid * NC + cid

    _zero_rows(rows[0], CH // 16, d)
    _zero_share(rows[0], acc, sid, rpt, d)
    plsc.subcore_barrier()

    def pipeline(start, count):
      ngrp = count // GRP

      def stage(g):
        row0 = wid * c0 + start + g * GRP
        s = g % 2
        pltpu.sync_copy(src_hbm.at[pl.ds(row0, GRP)], sidx.at[s])
        pltpu.sync_copy(dst_hbm.at[pl.ds(row0, GRP)], didx.at[s])

      def fire_gather(j):
        g, k = divmod(j, GRP)
        p = j % NBUF
        return pltpu.async_copy(h_hbm.at[sidx.at[g % 2].at[k]], rows[p],
                                gsem[p])

      def fire_scatter(j):
        g, k = divmod(j, GRP)
        p = j % NBUF
        return pltpu.async_copy(rows[p], acc.at[didx.at[g % 2].at[k]],
                                ssem[p], add=True)

      stage(0)
      pend_g = fire_gather(0)
      pend_s = [None] * NBUF
      for j in range(count):
        p = j % NBUF
        pend_g.wait()
        pend_s[p] = fire_scatter(j)
        if j + 1 < count:
          q = (j + 1) % NBUF
          if pend_s[q] is not None:
            pend_s[q].wait()
            pend_s[q] = None
          pend_g = fire_gather(j + 1)
        if j % GRP == 0 and j // GRP + 1 < ngrp:
          stage(j // GRP + 1)
      for p in range(NBUF):
        if pend_s[p] is not None:
          pend_s[p].wait()

    pipeline(0, c1)

    @pl.when(cid == 0)
    def _core0_extra():
      pipeline(c1, c0 - c1)

    plsc.subcore_barrier()
    sl = pl.ds(sid * rpt, rpt)
    osl = pl.ds(cid * n_pad + sid * rpt, rpt)
    pltpu.sync_copy(acc.at[sl], part_hbm.at[osl])

  return pl.kernel(body, out_type=out_type, mesh=mesh, scratch_types=scratch)


def _make_deg_pass(n_pad, d, nchunks):
  rpt = n_pad // NS
  ngrp = nchunks // GRP
  mesh = plsc.VectorSubcoreMesh(core_axis_name="c", subcore_axis_name="s")

  out_type = jax.ShapeDtypeStruct((NC * n_pad, d), jnp.float32)
  scratch = [
      pltpu.VMEM((GRP, CH), jnp.int32),
      pltpu.VMEM((CH, d), jnp.float32),
      pltpu.VMEM_SHARED((n_pad, d), jnp.float32),
  ]

  def body(dst_hbm, z_hbm, ones_hbm, deg_hbm, didx, ones_v, dacc):
    cid = lax.axis_index("c")
    sid = lax.axis_index("s")
    wid = sid * NC + cid

    pltpu.sync_copy(z_hbm, dacc.at[pl.ds(sid * rpt, rpt)])
    pltpu.sync_copy(ones_hbm, ones_v)
    plsc.subcore_barrier()

    @pl.loop(0, ngrp)
    def _grp(g):
      pltpu.sync_copy(dst_hbm.at[pl.ds(wid * nchunks + g * GRP, GRP)], didx)
      for j in range(GRP):
        pltpu.sync_copy(ones_v, dacc.at[didx.at[j]], add=True)

    plsc.subcore_barrier()
    sl = pl.ds(sid * rpt, rpt)
    pltpu.sync_copy(dacc.at[sl], deg_hbm.at[pl.ds(cid * n_pad + sid * rpt,
                                                  rpt)])

  return pl.kernel(body, out_type=out_type, mesh=mesh, scratch_types=scratch)


def _make_gather_pass(d, kch):
  mesh = plsc.VectorSubcoreMesh(core_axis_name="c", subcore_axis_name="s")
  out_type = jax.ShapeDtypeStruct((NW * kch * CH, d), jnp.float32)
  scratch = [
      pltpu.VMEM((kch, CH), jnp.int32),
      pltpu.VMEM((CH, d), jnp.float32),
      pltpu.SemaphoreType.DMA,
  ]

  def body(h_hbm, ids_hbm, out_hbm, idx_v, rows, sem):
    cid = lax.axis_index("c")
    sid = lax.axis_index("s")
    wid = sid * NC + cid
    pltpu.sync_copy(ids_hbm.at[pl.ds(wid * kch, kch)], idx_v)
    for j in range(kch):
      pltpu.async_copy(h_hbm.at[idx_v.at[j]], rows, sem).wait()
      pltpu.sync_copy(rows, out_hbm.at[pl.ds((wid * kch + j) * CH, CH)])

  return pl.kernel(body, out_type=out_type, mesh=mesh, scratch_types=scratch)


def _post_layer(part, dpart, h_in, w):
  n, d = h_in.shape
  blk = 1000
  grid = (n // blk,)

  def body(part_ref, dpart_ref, h_ref, w_ref, out_ref):
    p = part_ref[0] + part_ref[1]
    deg = dpart_ref[0][:, :1] + dpart_ref[1][:, :1]
    agg = p / jnp.maximum(deg, 1.0)
    y = lax.dot_general(agg, w_ref[...], (((1,), (1,)), ((), ())),
                        preferred_element_type=jnp.float32)
    y = jnp.maximum(y, 0.0)
    mu = jnp.mean(y, axis=-1, keepdims=True)
    var = jnp.mean((y - mu) * (y - mu), axis=-1, keepdims=True)
    out_ref[...] = (y - mu) * lax.rsqrt(var + 1e-5) + h_ref[...]

  return pl.pallas_call(
      body,
      grid=grid,
      in_specs=[
          pl.BlockSpec((NC, blk, d), lambda i: (0, i, 0)),
          pl.BlockSpec((NC, blk, d), lambda i: (0, i, 0)),
          pl.BlockSpec((blk, d), lambda i: (i, 0)),
          pl.BlockSpec((d, d), lambda i: (0, 0)),
      ],
      out_specs=pl.BlockSpec((blk, d), lambda i: (i, 0)),
      out_shape=jax.ShapeDtypeStruct((n, d), jnp.float32),
  )(part, dpart, h_in, w)


def _final_dot(aver_feats, w_t, emb2):
  b, d = aver_feats.shape
  blk = 512
  grid = (b // blk,)

  def body(a_ref, wt_ref, e_ref, out_ref):
    we = lax.dot_general(a_ref[...], wt_ref[...], (((1,), (1,)), ((), ())),
                         preferred_element_type=jnp.float32)
    e = e_ref[...]
    o0 = jnp.sum(e[:, :d] * we, axis=1, keepdims=True)
    o1 = jnp.sum(e[:, d:] * we, axis=1, keepdims=True)
    out_ref[...] = jnp.concatenate([o0, o1], axis=1)

  return pl.pallas_call(
      body,
      grid=grid,
      in_specs=[
          pl.BlockSpec((blk, d), lambda i: (i, 0)),
          pl.BlockSpec((d, d), lambda i: (0, 0)),
          pl.BlockSpec((blk, 2 * d), lambda i: (i, 0)),
      ],
      out_specs=pl.BlockSpec((blk, 2), lambda i: (i, 0)),
      out_shape=jax.ShapeDtypeStruct((b, 2), jnp.float32),
  )(aver_feats, w_t, emb2)


def kernel(node_table, aver_feats, W_transform, W1, W2, id_pairs, edge_index):
  n_nodes, d = node_table.shape
  b = aver_feats.shape[0]
  e = edge_index.shape[1]

  n_pad = ((n_nodes + 1 + NS - 1) // NS + 7) // 8 * 8 * NS
  nch_real = -(-e // CH)
  nch_eq = -(-nch_real // (NW * GRP)) * GRP
  npad_ch = NW * nch_eq
  e_pad = npad_ch * CH

  src = edge_index[0].astype(jnp.int32)
  dst = edge_index[1].astype(jnp.int32)
  src_c = jnp.concatenate(
      [src, jnp.zeros((e_pad - e,), jnp.int32)]).reshape(npad_ch, CH)
  dst_c = jnp.concatenate(
      [dst, jnp.full((e_pad - e,), n_nodes, jnp.int32)]).reshape(npad_ch, CH)

  c1 = max(GRP, int(round(nch_real * 0.25 / (NS * GRP))) * GRP)
  c0 = -(-(npad_ch - NS * c1) // (NS * GRP)) * GRP
  perm = np.zeros((NW, c0), np.int32)
  for w in range(NW):
    if w % NC == 0:
      perm[w, :] = (w // NC) * c0 + np.arange(c0)
    else:
      perm[w, :c1] = NS * c0 + (w // NC) * c1 + np.arange(c1)
      perm[w, c1:] = npad_ch - 1
  perm = jnp.asarray(perm.reshape(-1))
  src_b = jnp.take(src_c, perm, axis=0)
  dst_b = jnp.take(dst_c, perm, axis=0)

  rpt = n_pad // NS
  z_hbm = jnp.zeros((rpt, d), jnp.float32)
  ones_hbm = jnp.ones((CH, d), jnp.float32)

  edge_pass = _make_edge_pass(n_nodes, n_pad, d, c0, c1)

  dpart = _make_deg_pass(n_pad, d, nch_eq)(dst_c, z_hbm, ones_hbm)
  dpart = dpart.reshape(NC, n_pad, d)
  part1 = edge_pass(node_table, src_b, dst_b).reshape(NC, n_pad, d)
  h1 = _post_layer(part1, dpart, node_table, W1)
  part2 = edge_pass(h1, src_b, dst_b).reshape(NC, n_pad, d)
  h2 = _post_layer(part2, dpart, h1, W2)

  flat = id_pairs.reshape(-1).astype(jnp.int32)
  kch = flat.shape[0] // (NW * CH)
  ids_r = flat.reshape(NW * kch, CH)
  emb = _make_gather_pass(d, kch)(h2, ids_r)
  emb2 = emb.reshape(b, 2 * d)

  return _final_dot(aver_feats, W_transform, emb2)


# speedup vs baseline: 1.2123x; 1.2123x over previous
import jax
import jax.numpy as jnp
import numpy as np
from jax import lax
from jax.experimental import pallas as pl
from jax.experimental.pallas import tpu as pltpu
from jax.experimental.pallas import tpu_sc as plsc

NC = 2
NS = 16
NW = NC * NS
CH = 128


GRP = 8


NBUF = 2


def _zero_rows(buf, n_sub, d):
  @pl.loop(0, n_sub * 16)
  def _row(i):
    for k in range(d // 16):
      buf[i, pl.ds(k * 16, 16)] = jnp.zeros((16,), jnp.float32)


def _zero_share(buf, acc, sid, rpt, d):
  nfull, tail = divmod(rpt, CH)
  for t in range(nfull):
    pltpu.sync_copy(buf, acc.at[pl.ds(sid * rpt + t * CH, CH)])
  if tail:
    pltpu.sync_copy(buf.at[pl.ds(0, tail)],
                    acc.at[pl.ds(sid * rpt + nfull * CH, tail)])


def _make_edge_pass(n_nodes, n_pad, d, c0, c1):
  rpt = n_pad // NS
  mesh = plsc.VectorSubcoreMesh(core_axis_name="c", subcore_axis_name="s")

  out_type = jax.ShapeDtypeStruct((NC * n_pad, d), jnp.float32)

  scratch = [
      pltpu.VMEM((2, GRP, CH), jnp.int32),
      pltpu.VMEM((2, GRP, CH), jnp.int32),
      [pltpu.VMEM((CH, d), jnp.float32) for _ in range(NBUF)],
      pltpu.VMEM_SHARED((n_pad, d), jnp.float32),
      [pltpu.SemaphoreType.DMA for _ in range(NBUF)],
      [pltpu.SemaphoreType.DMA for _ in range(NBUF)],
  ]

  def body(h_hbm, src_hbm, dst_hbm, part_hbm, sidx, didx, rows, acc, gsem,
           ssem):
    cid = lax.axis_index("c")
    sid = lax.axis_index("s")

    _zero_rows(rows[0], CH // 16, d)
    _zero_share(rows[0], acc, sid, rpt, d)
    plsc.subcore_barrier()

    base = jnp.where(cid == 0, sid * c0, NS * c0 + sid * c1)

    def pipeline(start, count):
      ngrp = count // GRP

      def stage(g):
        row0 = base + start + g * GRP
        s = g % 2
        pltpu.sync_copy(src_hbm.at[pl.ds(row0, GRP)], sidx.at[s])
        pltpu.sync_copy(dst_hbm.at[pl.ds(row0, GRP)], didx.at[s])

      def fire_gather(j):
        g, k = divmod(j, GRP)
        p = j % NBUF
        return pltpu.async_copy(h_hbm.at[sidx.at[g % 2].at[k]], rows[p],
                                gsem[p])

      def fire_scatter(j):
        g, k = divmod(j, GRP)
        p = j % NBUF
        return pltpu.async_copy(rows[p], acc.at[didx.at[g % 2].at[k]],
                                ssem[p], add=True)

      stage(0)
      pend_g = fire_gather(0)
      pend_s = [None] * NBUF
      for j in range(count):
        p = j % NBUF
        pend_g.wait()
        pend_s[p] = fire_scatter(j)
        if j + 1 < count:
          q = (j + 1) % NBUF
          if pend_s[q] is not None:
            pend_s[q].wait()
            pend_s[q] = None
          pend_g = fire_gather(j + 1)
        if j % GRP == 0 and j // GRP + 1 < ngrp:
          stage(j // GRP + 1)
      for p in range(NBUF):
        if pend_s[p] is not None:
          pend_s[p].wait()

    pipeline(0, c1)

    @pl.when(cid == 0)
    def _core0_extra():
      pipeline(c1, c0 - c1)

    plsc.subcore_barrier()
    sl = pl.ds(sid * rpt, rpt)
    osl = pl.ds(cid * n_pad + sid * rpt, rpt)
    pltpu.sync_copy(acc.at[sl], part_hbm.at[osl])

  return pl.kernel(body, out_type=out_type, mesh=mesh, scratch_types=scratch)


def _make_deg_pass(n_pad, d, nchunks):
  rpt = n_pad // NS
  ngrp = nchunks // GRP
  mesh = plsc.VectorSubcoreMesh(core_axis_name="c", subcore_axis_name="s")

  out_type = jax.ShapeDtypeStruct((NC * n_pad, d), jnp.float32)
  scratch = [
      pltpu.VMEM((GRP, CH), jnp.int32),
      pltpu.VMEM((CH, d), jnp.float32),
      pltpu.VMEM_SHARED((n_pad, d), jnp.float32),
  ]

  def body(dst_hbm, z_hbm, ones_hbm, deg_hbm, didx, ones_v, dacc):
    cid = lax.axis_index("c")
    sid = lax.axis_index("s")
    wid = sid * NC + cid

    pltpu.sync_copy(z_hbm, dacc.at[pl.ds(sid * rpt, rpt)])
    pltpu.sync_copy(ones_hbm, ones_v)
    plsc.subcore_barrier()

    @pl.loop(0, ngrp)
    def _grp(g):
      pltpu.sync_copy(dst_hbm.at[pl.ds(wid * nchunks + g * GRP, GRP)], didx)
      for j in range(GRP):
        pltpu.sync_copy(ones_v, dacc.at[didx.at[j]], add=True)

    plsc.subcore_barrier()
    sl = pl.ds(sid * rpt, rpt)
    pltpu.sync_copy(dacc.at[sl], deg_hbm.at[pl.ds(cid * n_pad + sid * rpt,
                                                  rpt)])

  return pl.kernel(body, out_type=out_type, mesh=mesh, scratch_types=scratch)


def _make_gather_pass(d, kch):
  mesh = plsc.VectorSubcoreMesh(core_axis_name="c", subcore_axis_name="s")
  out_type = jax.ShapeDtypeStruct((NW * kch * CH, d), jnp.float32)
  scratch = [
      pltpu.VMEM((kch, CH), jnp.int32),
      pltpu.VMEM((CH, d), jnp.float32),
      pltpu.SemaphoreType.DMA,
  ]

  def body(h_hbm, ids_hbm, out_hbm, idx_v, rows, sem):
    cid = lax.axis_index("c")
    sid = lax.axis_index("s")
    wid = sid * NC + cid
    pltpu.sync_copy(ids_hbm.at[pl.ds(wid * kch, kch)], idx_v)
    for j in range(kch):
      pltpu.async_copy(h_hbm.at[idx_v.at[j]], rows, sem).wait()
      pltpu.sync_copy(rows, out_hbm.at[pl.ds((wid * kch + j) * CH, CH)])

  return pl.kernel(body, out_type=out_type, mesh=mesh, scratch_types=scratch)


def _post_layer(part, dpart, h_in, w):
  n, d = h_in.shape
  blk = 1000
  grid = (n // blk,)

  def body(part_ref, dpart_ref, h_ref, w_ref, out_ref):
    p = part_ref[0] + part_ref[1]
    deg = dpart_ref[0][:, :1] + dpart_ref[1][:, :1]
    agg = p / jnp.maximum(deg, 1.0)
    y = lax.dot_general(agg, w_ref[...], (((1,), (1,)), ((), ())),
                        preferred_element_type=jnp.float32)
    y = jnp.maximum(y, 0.0)
    mu = jnp.mean(y, axis=-1, keepdims=True)
    var = jnp.mean((y - mu) * (y - mu), axis=-1, keepdims=True)
    out_ref[...] = (y - mu) * lax.rsqrt(var + 1e-5) + h_ref[...]

  return pl.pallas_call(
      body,
      grid=grid,
      in_specs=[
          pl.BlockSpec((NC, blk, d), lambda i: (0, i, 0)),
          pl.BlockSpec((NC, blk, d), lambda i: (0, i, 0)),
          pl.BlockSpec((blk, d), lambda i: (i, 0)),
          pl.BlockSpec((d, d), lambda i: (0, 0)),
      ],
      out_specs=pl.BlockSpec((blk, d), lambda i: (i, 0)),
      out_shape=jax.ShapeDtypeStruct((n, d), jnp.float32),
  )(part, dpart, h_in, w)


def _final_dot(aver_feats, w_t, emb2):
  b, d = aver_feats.shape
  blk = 512
  grid = (b // blk,)

  def body(a_ref, wt_ref, e_ref, out_ref):
    we = lax.dot_general(a_ref[...], wt_ref[...], (((1,), (1,)), ((), ())),
                         preferred_element_type=jnp.float32)
    e = e_ref[...]
    o0 = jnp.sum(e[:, :d] * we, axis=1, keepdims=True)
    o1 = jnp.sum(e[:, d:] * we, axis=1, keepdims=True)
    out_ref[...] = jnp.concatenate([o0, o1], axis=1)

  return pl.pallas_call(
      body,
      grid=grid,
      in_specs=[
          pl.BlockSpec((blk, d), lambda i: (i, 0)),
          pl.BlockSpec((d, d), lambda i: (0, 0)),
          pl.BlockSpec((blk, 2 * d), lambda i: (i, 0)),
      ],
      out_specs=pl.BlockSpec((blk, 2), lambda i: (i, 0)),
      out_shape=jax.ShapeDtypeStruct((b, 2), jnp.float32),
  )(aver_feats, w_t, emb2)


def kernel(node_table, aver_feats, W_transform, W1, W2, id_pairs, edge_index):
  n_nodes, d = node_table.shape
  b = aver_feats.shape[0]
  e = edge_index.shape[1]

  n_pad = ((n_nodes + 1 + NS - 1) // NS + 7) // 8 * 8 * NS
  nch_real = -(-e // CH)
  nch_eq = -(-nch_real // (NW * GRP)) * GRP
  npad_ch = NW * nch_eq
  e_pad = npad_ch * CH

  src = edge_index[0].astype(jnp.int32)
  dst = edge_index[1].astype(jnp.int32)
  src_c = jnp.concatenate(
      [src, jnp.zeros((e_pad - e,), jnp.int32)]).reshape(npad_ch, CH)
  dst_c = jnp.concatenate(
      [dst, jnp.full((e_pad - e,), n_nodes, jnp.int32)]).reshape(npad_ch, CH)

  c1 = max(GRP, int(round(nch_real * 0.25 / (NS * GRP))) * GRP)
  c0 = -(-(npad_ch - NS * c1) // (NS * GRP)) * GRP

  rpt = n_pad // NS
  z_hbm = jnp.zeros((rpt, d), jnp.float32)
  ones_hbm = jnp.ones((CH, d), jnp.float32)

  edge_pass = _make_edge_pass(n_nodes, n_pad, d, c0, c1)

  dpart = _make_deg_pass(n_pad, d, nch_eq)(dst_c, z_hbm, ones_hbm)
  dpart = dpart.reshape(NC, n_pad, d)
  part1 = edge_pass(node_table, src_c, dst_c).reshape(NC, n_pad, d)
  h1 = _post_layer(part1, dpart, node_table, W1)
  part2 = edge_pass(h1, src_c, dst_c).reshape(NC, n_pad, d)
  h2 = _post_layer(part2, dpart, h1, W2)

  flat = id_pairs.reshape(-1).astype(jnp.int32)
  kch = flat.shape[0] // (NW * CH)
  ids_r = flat.reshape(NW * kch, CH)
  emb = _make_gather_pass(d, kch)(h2, ids_r)
  emb2 = emb.reshape(b, 2 * d)

  return _final_dot(aver_feats, W_transform, emb2)


# speedup vs baseline: 1.2756x; 1.0522x over previous
import jax
import jax.numpy as jnp
import numpy as np
from jax import lax
from jax.experimental import pallas as pl
from jax.experimental.pallas import tpu as pltpu
from jax.experimental.pallas import tpu_sc as plsc

NC = 2
NS = 16
NW = NC * NS
CH = 128


GRP = 8


NBUF = 2


def _zero_rows(buf, n_sub, d):
  @pl.loop(0, n_sub * 16)
  def _row(i):
    for k in range(d // 16):
      buf[i, pl.ds(k * 16, 16)] = jnp.zeros((16,), jnp.float32)


def _zero_share(buf, acc, sid, rpt, d):
  nfull, tail = divmod(rpt, CH)
  for t in range(nfull):
    pltpu.sync_copy(buf, acc.at[pl.ds(sid * rpt + t * CH, CH)])
  if tail:
    pltpu.sync_copy(buf.at[pl.ds(0, tail)],
                    acc.at[pl.ds(sid * rpt + nfull * CH, tail)])


def _make_edge_pass(n_nodes, n_pad, d, c0, c1):
  rpt = n_pad // NS
  mesh = plsc.VectorSubcoreMesh(core_axis_name="c", subcore_axis_name="s")

  out_type = jax.ShapeDtypeStruct((NC * n_pad, d), jnp.float32)

  scratch = [
      pltpu.VMEM((2, GRP, CH), jnp.int32),
      pltpu.VMEM((2, GRP, CH), jnp.int32),
      [pltpu.VMEM((CH, d), jnp.float32) for _ in range(NBUF)],
      pltpu.VMEM_SHARED((n_pad, d), jnp.float32),
      [pltpu.SemaphoreType.DMA for _ in range(NBUF)],
      [pltpu.SemaphoreType.DMA for _ in range(NBUF)],
  ]

  def body(h_hbm, src_hbm, dst_hbm, part_hbm, sidx, didx, rows, acc, gsem,
           ssem):
    cid = lax.axis_index("c")
    sid = lax.axis_index("s")

    _zero_rows(rows[0], CH // 16, d)
    _zero_share(rows[0], acc, sid, rpt, d)
    plsc.subcore_barrier()

    base = jnp.where(cid == 0, sid * c0, NS * c0 + sid * c1)

    def pipeline(start, count):
      ngrp = count // GRP

      def stage(g):
        row0 = base + start + g * GRP
        s = g % 2
        pltpu.sync_copy(src_hbm.at[pl.ds(row0, GRP)], sidx.at[s])
        pltpu.sync_copy(dst_hbm.at[pl.ds(row0, GRP)], didx.at[s])

      def fire_gather(j):
        g, k = divmod(j, GRP)
        p = j % NBUF
        return pltpu.async_copy(h_hbm.at[sidx.at[g % 2].at[k]], rows[p],
                                gsem[p])

      def fire_scatter(j):
        g, k = divmod(j, GRP)
        p = j % NBUF
        return pltpu.async_copy(rows[p], acc.at[didx.at[g % 2].at[k]],
                                ssem[p], add=True)

      stage(0)
      pend_g = fire_gather(0)
      pend_s = [None] * NBUF
      for j in range(count):
        p = j % NBUF
        pend_g.wait()
        pend_s[p] = fire_scatter(j)
        if j + 1 < count:
          q = (j + 1) % NBUF
          if pend_s[q] is not None:
            pend_s[q].wait()
            pend_s[q] = None
          pend_g = fire_gather(j + 1)
        if j % GRP == 0 and j // GRP + 1 < ngrp:
          stage(j // GRP + 1)
      for p in range(NBUF):
        if pend_s[p] is not None:
          pend_s[p].wait()

    pipeline(0, c1)

    @pl.when(cid == 0)
    def _core0_extra():
      pipeline(c1, c0 - c1)

    plsc.subcore_barrier()
    sl = pl.ds(sid * rpt, rpt)
    osl = pl.ds(cid * n_pad + sid * rpt, rpt)
    pltpu.sync_copy(acc.at[sl], part_hbm.at[osl])

  return pl.kernel(body, out_type=out_type, mesh=mesh, scratch_types=scratch)


def _make_deg_pass(n_pad, d, nchunks):
  rpt = n_pad // NS
  ngrp = nchunks // GRP
  mesh = plsc.VectorSubcoreMesh(core_axis_name="c", subcore_axis_name="s")

  out_type = jax.ShapeDtypeStruct((NC * n_pad, d), jnp.float32)
  scratch = [
      pltpu.VMEM((GRP, CH), jnp.int32),
      pltpu.VMEM((CH, d), jnp.float32),
      pltpu.VMEM_SHARED((n_pad, d), jnp.float32),
  ]

  def body(dst_hbm, z_hbm, ones_hbm, deg_hbm, didx, ones_v, dacc):
    cid = lax.axis_index("c")
    sid = lax.axis_index("s")
    wid = sid * NC + cid

    pltpu.sync_copy(z_hbm, dacc.at[pl.ds(sid * rpt, rpt)])
    pltpu.sync_copy(ones_hbm, ones_v)
    plsc.subcore_barrier()

    @pl.loop(0, ngrp)
    def _grp(g):
      pltpu.sync_copy(dst_hbm.at[pl.ds(wid * nchunks + g * GRP, GRP)], didx)
      for j in range(GRP):
        pltpu.sync_copy(ones_v, dacc.at[didx.at[j]], add=True)

    plsc.subcore_barrier()
    sl = pl.ds(sid * rpt, rpt)
    pltpu.sync_copy(dacc.at[sl], deg_hbm.at[pl.ds(cid * n_pad + sid * rpt,
                                                  rpt)])

  return pl.kernel(body, out_type=out_type, mesh=mesh, scratch_types=scratch)


def _make_gather_pass(d, kch):
  mesh = plsc.VectorSubcoreMesh(core_axis_name="c", subcore_axis_name="s")
  out_type = jax.ShapeDtypeStruct((NW * kch * CH, d), jnp.float32)
  scratch = [
      pltpu.VMEM((kch, CH), jnp.int32),
      pltpu.VMEM((CH, d), jnp.float32),
      pltpu.SemaphoreType.DMA,
  ]

  def body(h_hbm, ids_hbm, out_hbm, idx_v, rows, sem):
    cid = lax.axis_index("c")
    sid = lax.axis_index("s")
    wid = sid * NC + cid
    pltpu.sync_copy(ids_hbm.at[pl.ds(wid * kch, kch)], idx_v)
    for j in range(kch):
      pltpu.async_copy(h_hbm.at[idx_v.at[j]], rows, sem).wait()
      pltpu.sync_copy(rows, out_hbm.at[pl.ds((wid * kch + j) * CH, CH)])

  return pl.kernel(body, out_type=out_type, mesh=mesh, scratch_types=scratch)


def _post_layer(part, dpart, h_in, w):
  n, d = h_in.shape
  blk = 1000
  grid = (n // blk,)

  def body(part_ref, dpart_ref, h_ref, w_ref, out_ref):
    p = part_ref[0] + part_ref[1]
    deg = dpart_ref[0][:, :1] + dpart_ref[1][:, :1]
    agg = p / jnp.maximum(deg, 1.0)
    y = lax.dot_general(agg, w_ref[...], (((1,), (1,)), ((), ())),
                        preferred_element_type=jnp.float32)
    y = jnp.maximum(y, 0.0)
    mu = jnp.mean(y, axis=-1, keepdims=True)
    var = jnp.mean((y - mu) * (y - mu), axis=-1, keepdims=True)
    out_ref[...] = (y - mu) * lax.rsqrt(var + 1e-5) + h_ref[...]

  return pl.pallas_call(
      body,
      grid=grid,
      in_specs=[
          pl.BlockSpec((NC, blk, d), lambda i: (0, i, 0)),
          pl.BlockSpec((NC, blk, d), lambda i: (0, i, 0)),
          pl.BlockSpec((blk, d), lambda i: (i, 0)),
          pl.BlockSpec((d, d), lambda i: (0, 0)),
      ],
      out_specs=pl.BlockSpec((blk, d), lambda i: (i, 0)),
      out_shape=jax.ShapeDtypeStruct((n, d), jnp.float32),
  )(part, dpart, h_in, w)


def _final_dot(aver_feats, w_t, emb2):
  b, d = aver_feats.shape
  blk = 512
  grid = (b // blk,)

  def body(a_ref, wt_ref, e_ref, out_ref):
    we = lax.dot_general(a_ref[...], wt_ref[...], (((1,), (1,)), ((), ())),
                         preferred_element_type=jnp.float32)
    e = e_ref[...]
    o0 = jnp.sum(e[:, :d] * we, axis=1, keepdims=True)
    o1 = jnp.sum(e[:, d:] * we, axis=1, keepdims=True)
    out_ref[...] = jnp.concatenate([o0, o1], axis=1)

  return pl.pallas_call(
      body,
      grid=grid,
      in_specs=[
          pl.BlockSpec((blk, d), lambda i: (i, 0)),
          pl.BlockSpec((d, d), lambda i: (0, 0)),
          pl.BlockSpec((blk, 2 * d), lambda i: (i, 0)),
      ],
      out_specs=pl.BlockSpec((blk, 2), lambda i: (i, 0)),
      out_shape=jax.ShapeDtypeStruct((b, 2), jnp.float32),
  )(aver_feats, w_t, emb2)


def kernel(node_table, aver_feats, W_transform, W1, W2, id_pairs, edge_index):
  n_nodes, d = node_table.shape
  b = aver_feats.shape[0]
  e = edge_index.shape[1]

  n_pad = ((n_nodes + 1 + NS - 1) // NS + 7) // 8 * 8 * NS
  nch_real = -(-e // CH)
  nch_eq = -(-nch_real // (NW * GRP)) * GRP
  npad_ch = NW * nch_eq
  e_pad = npad_ch * CH

  src = edge_index[0].astype(jnp.int32)
  dst = edge_index[1].astype(jnp.int32)
  src_c = jnp.concatenate(
      [src, jnp.zeros((e_pad - e,), jnp.int32)]).reshape(npad_ch, CH)
  dst_c = jnp.concatenate(
      [dst, jnp.full((e_pad - e,), n_nodes, jnp.int32)]).reshape(npad_ch, CH)

  c1 = GRP
  c0 = -(-(npad_ch - NS * c1) // (NS * GRP)) * GRP

  rpt = n_pad // NS
  z_hbm = jnp.zeros((rpt, d), jnp.float32)
  ones_hbm = jnp.ones((CH, d), jnp.float32)

  edge_pass = _make_edge_pass(n_nodes, n_pad, d, c0, c1)

  dpart = _make_deg_pass(n_pad, d, nch_eq)(dst_c, z_hbm, ones_hbm)
  dpart = dpart.reshape(NC, n_pad, d)
  part1 = edge_pass(node_table, src_c, dst_c).reshape(NC, n_pad, d)
  h1 = _post_layer(part1, dpart, node_table, W1)
  part2 = edge_pass(h1, src_c, dst_c).reshape(NC, n_pad, d)
  h2 = _post_layer(part2, dpart, h1, W2)

  flat = id_pairs.reshape(-1).astype(jnp.int32)
  kch = flat.shape[0] // (NW * CH)
  ids_r = flat.reshape(NW * kch, CH)
  emb = _make_gather_pass(d, kch)(h2, ids_r)
  emb2 = emb.reshape(b, 2 * d)

  return _final_dot(aver_feats, W_transform, emb2)
